# baseline (speedup 1.0000x reference)
import functools

import jax
import jax.numpy as jnp
from jax import lax
from jax.experimental import pallas as pl
from jax.experimental.pallas import tpu as pltpu
from jax.experimental.pallas import tpu_sc as plsc

BATCH = 4
SEQ_LEN = 2048
EMBED_DIM = 768

NC = 2
NS = 16
NW = NC * NS

ROWS = BATCH * SEQ_LEN
FLAT = ROWS * EMBED_DIM
PER_W = FLAT // NW
POS_FLAT = SEQ_LEN * EMBED_DIM
CHUNK = 24576
NCHUNK = PER_W // CHUNK
LANES = 16

_mesh = plsc.VectorSubcoreMesh(core_axis_name="c", subcore_axis_name="s")


@functools.partial(
    pl.kernel,
    out_type=jax.ShapeDtypeStruct((FLAT,), jnp.float32),
    mesh=_mesh,
    scratch_types=[
        pltpu.VMEM((CHUNK,), jnp.float32),
        pltpu.VMEM((CHUNK,), jnp.float32),
    ],
)
def _sc_pos_add(x_hbm, pos_hbm, out_hbm, xbuf, pbuf):
    wid = lax.axis_index("s") * NC + lax.axis_index("c")
    base = wid * PER_W
    p_base = lax.rem(base, POS_FLAT)

    def chunk(c, carry):
        off = c * CHUNK
        pltpu.sync_copy(x_hbm.at[pl.ds(base + off, CHUNK)], xbuf)
        pltpu.sync_copy(pos_hbm.at[pl.ds(p_base + off, CHUNK)], pbuf)

        def add16(i, carry2):
            sl = pl.ds(i * LANES, LANES)
            xbuf[sl] = xbuf[sl] + pbuf[sl]
            return carry2

        lax.fori_loop(0, CHUNK // LANES, add16, 0)
        pltpu.sync_copy(xbuf, out_hbm.at[pl.ds(base + off, CHUNK)])
        return carry

    lax.fori_loop(0, NCHUNK, chunk, 0)


def kernel(x, pos_table):
    out = _sc_pos_add(x.reshape(FLAT), pos_table.reshape(POS_FLAT))
    return out.reshape(BATCH, SEQ_LEN, EMBED_DIM)


# speedup vs baseline: 1.4818x; 1.4818x over previous
import functools

import jax
import jax.numpy as jnp
from jax import lax
from jax.experimental import pallas as pl
from jax.experimental.pallas import tpu as pltpu
from jax.experimental.pallas import tpu_sc as plsc

BATCH = 4
SEQ_LEN = 2048
EMBED_DIM = 768

NC = 2
NS = 16
NW = NC * NS

ROWS = BATCH * SEQ_LEN
FLAT = ROWS * EMBED_DIM
PER_W = FLAT // NW
POS_FLAT = SEQ_LEN * EMBED_DIM
CHUNK = 24576
NCHUNK = PER_W // CHUNK
LANES = 16

_mesh = plsc.VectorSubcoreMesh(core_axis_name="c", subcore_axis_name="s")


@functools.partial(
    pl.kernel,
    out_type=jax.ShapeDtypeStruct((FLAT,), jnp.float32),
    mesh=_mesh,
    scratch_types=[
        pltpu.VMEM((2, CHUNK), jnp.float32),
        pltpu.VMEM((2, CHUNK), jnp.float32),
        pltpu.SemaphoreType.DMA,
        pltpu.SemaphoreType.DMA,
        pltpu.SemaphoreType.DMA,
        pltpu.SemaphoreType.DMA,
        pltpu.SemaphoreType.DMA,
        pltpu.SemaphoreType.DMA,
    ],
)
def _sc_pos_add(x_hbm, pos_hbm, out_hbm, xb, pb, sx0, sx1, sp0, sp1, so0, so1):
    wid = lax.axis_index("s") * NC + lax.axis_index("c")
    base = wid * PER_W
    p_base = lax.rem(base, POS_FLAT)
    sx = (sx0, sx1)
    sp = (sp0, sp1)
    so = (so0, so1)

    def load(c):
        k = c % 2
        off = c * CHUNK
        pltpu.async_copy(x_hbm.at[pl.ds(base + off, CHUNK)], xb.at[k], sx[k])
        pltpu.async_copy(pos_hbm.at[pl.ds(p_base + off, CHUNK)], pb.at[k], sp[k])

    load(0)
    for c in range(NCHUNK):
        k = c % 2
        off = c * CHUNK
        pltpu.make_async_copy(x_hbm.at[pl.ds(base + off, CHUNK)], xb.at[k], sx[k]).wait()
        pltpu.make_async_copy(pos_hbm.at[pl.ds(p_base + off, CHUNK)], pb.at[k], sp[k]).wait()
        if c + 1 < NCHUNK:
            if c >= 1:
                po = (c - 1) * CHUNK
                pltpu.make_async_copy(
                    xb.at[1 - k], out_hbm.at[pl.ds(base + po, CHUNK)], so[1 - k]
                ).wait()
            load(c + 1)

        @plsc.parallel_loop(0, CHUNK, step=LANES, unroll=8)
        def _add(i):
            xb[k, pl.ds(i, LANES)] = xb[k, pl.ds(i, LANES)] + pb[k, pl.ds(i, LANES)]

        pltpu.async_copy(xb.at[k], out_hbm.at[pl.ds(base + off, CHUNK)], so[k])

    for c in (NCHUNK - 2, NCHUNK - 1):
        k = c % 2
        off = c * CHUNK
        pltpu.make_async_copy(xb.at[k], out_hbm.at[pl.ds(base + off, CHUNK)], so[k]).wait()


def kernel(x, pos_table):
    out = _sc_pos_add(x.reshape(FLAT), pos_table.reshape(POS_FLAT))
    return out.reshape(BATCH, SEQ_LEN, EMBED_DIM)
